# Initial kernel scaffold; baseline (speedup 1.0000x reference)
import jax
import jax.numpy as jnp
from jax.experimental import pallas as pl


def kernel(melody_tokens, chord_tokens, melody_emb, chord_emb, enc_pos, dec_pos):
    raise NotImplementedError("write your pallas kernel here")


# speedup vs baseline: 5.1088x; 5.1088x over previous
import functools

import jax
import jax.numpy as jnp
from jax import lax
from jax.experimental import pallas as pl
from jax.experimental.pallas import tpu as pltpu
from jax.experimental.pallas import tpu_sc as plsc

_D = 32
_SEQ = 200
_NW = 32
_CHUNK_SEQS = 4
_CHUNK = _CHUNK_SEQS * _SEQ


def _sc_body(mel_idx, chd_idx, mel_emb, chd_emb, mel_pos, chd_pos,
             mel_out, chd_out, idx_v, rows_v, pos_v, sem):
    cid = lax.axis_index("c")
    sid = lax.axis_index("s")
    wid = sid * 2 + cid
    total_rows = mel_idx.shape[0]
    rows_per_w = total_rows // _NW
    n_chunks = rows_per_w // _CHUNK
    base0 = wid * rows_per_w

    def run_table(idx_hbm, table_hbm, pos_hbm, out_hbm):
        pltpu.sync_copy(pos_hbm, pos_v)

        def chunk_body(g, carry):
            base = base0 + g * _CHUNK
            pltpu.sync_copy(idx_hbm.at[pl.ds(base, _CHUNK)], idx_v)
            pltpu.async_copy(table_hbm.at[idx_v], rows_v, sem).wait()

            def add_pos(s, c2):
                p0 = pos_v[s, pl.ds(0, 16)]
                p1 = pos_v[s, pl.ds(16, 16)]
                for j in range(_CHUNK_SEQS):
                    r = j * _SEQ + s
                    rows_v[r, pl.ds(0, 16)] = rows_v[r, pl.ds(0, 16)] + p0
                    rows_v[r, pl.ds(16, 16)] = rows_v[r, pl.ds(16, 16)] + p1
                return c2

            lax.fori_loop(0, _SEQ, add_pos, 0)
            pltpu.sync_copy(rows_v, out_hbm.at[pl.ds(base, _CHUNK)])
            return carry

        lax.fori_loop(0, n_chunks, chunk_body, 0)

    run_table(mel_idx, mel_emb, mel_pos, mel_out)
    run_table(chd_idx, chd_emb, chd_pos, chd_out)


def kernel(melody_tokens, chord_tokens, melody_emb, chord_emb, enc_pos, dec_pos):
    batch, seq = melody_tokens.shape
    total = batch * seq
    mel_idx = melody_tokens.reshape(total).astype(jnp.int32)
    chd_idx = chord_tokens.reshape(total).astype(jnp.int32)
    mel_pos = enc_pos[:seq]
    chd_pos = dec_pos[:seq]

    mesh = plsc.VectorSubcoreMesh(
        core_axis_name="c", subcore_axis_name="s", num_cores=2, num_subcores=16
    )
    run = pl.kernel(
        _sc_body,
        out_type=(
            jax.ShapeDtypeStruct((total, _D), jnp.float32),
            jax.ShapeDtypeStruct((total, _D), jnp.float32),
        ),
        mesh=mesh,
        scratch_types=[
            pltpu.VMEM((_CHUNK,), jnp.int32),
            pltpu.VMEM((_CHUNK, _D), jnp.float32),
            pltpu.VMEM((_SEQ, _D), jnp.float32),
            pltpu.SemaphoreType.DMA,
        ],
        compiler_params=pltpu.CompilerParams(use_tc_tiling_on_sc=False),
    )
    mel_out, chd_out = run(mel_idx, chd_idx, melody_emb, chord_emb, mel_pos, chd_pos)
    return (mel_out.reshape(batch, seq, _D), chd_out.reshape(batch, seq, _D))


# speedup vs baseline: 5.6392x; 1.1038x over previous
import functools

import jax
import jax.numpy as jnp
from jax import lax
from jax.experimental import pallas as pl
from jax.experimental.pallas import tpu as pltpu
from jax.experimental.pallas import tpu_sc as plsc

_D = 32
_SEQ = 200
_NW = 32
_CHUNK_SEQS = 4
_CHUNK = _CHUNK_SEQS * _SEQ


def _sc_body(mel_idx, chd_idx, mel_emb, chd_emb, mel_pos, chd_pos,
             mel_out, chd_out, idx_v0, idx_v1, rows_v0, rows_v1, pos_v,
             sem0, sem1):
    cid = lax.axis_index("c")
    sid = lax.axis_index("s")
    wid = sid * 2 + cid
    total_rows = mel_idx.shape[0]
    rows_per_w = total_rows // _NW
    n_chunks = rows_per_w // _CHUNK
    base0 = wid * rows_per_w
    idx_bufs = (idx_v0, idx_v1)
    rows_bufs = (rows_v0, rows_v1)
    sems = (sem0, sem1)

    def run_table(idx_hbm, table_hbm, pos_hbm, out_hbm):
        pltpu.sync_copy(pos_hbm, pos_v)

        def prefetch(g, half):
            pltpu.sync_copy(idx_hbm.at[pl.ds(base0 + g * _CHUNK, _CHUNK)],
                            idx_bufs[half])
            pltpu.async_copy(table_hbm.at[idx_bufs[half]], rows_bufs[half],
                             sems[half])

        def add_pos(rows_v):
            def body(s, c2):
                p0 = pos_v[s, pl.ds(0, 16)]
                p1 = pos_v[s, pl.ds(16, 16)]
                for j in range(_CHUNK_SEQS):
                    r = j * _SEQ + s
                    rows_v[r, pl.ds(0, 16)] = rows_v[r, pl.ds(0, 16)] + p0
                    rows_v[r, pl.ds(16, 16)] = rows_v[r, pl.ds(16, 16)] + p1
                return c2

            lax.fori_loop(0, _SEQ, body, 0)

        prefetch(0, 0)

        def chunk_pair(g2, carry):
            for half in range(2):
                g = 2 * g2 + half
                nxt = g + 1

                @pl.when(nxt < n_chunks)
                def _():
                    prefetch(nxt, 1 - half)

                pltpu.make_async_copy(table_hbm.at[idx_bufs[half]],
                                      rows_bufs[half], sems[half]).wait()
                add_pos(rows_bufs[half])
                pltpu.sync_copy(rows_bufs[half],
                                out_hbm.at[pl.ds(base0 + g * _CHUNK, _CHUNK)])
            return carry

        lax.fori_loop(0, n_chunks // 2, chunk_pair, 0)

    run_table(mel_idx, mel_emb, mel_pos, mel_out)
    run_table(chd_idx, chd_emb, chd_pos, chd_out)


def kernel(melody_tokens, chord_tokens, melody_emb, chord_emb, enc_pos, dec_pos):
    batch, seq = melody_tokens.shape
    total = batch * seq
    mel_idx = melody_tokens.reshape(total).astype(jnp.int32)
    chd_idx = chord_tokens.reshape(total).astype(jnp.int32)
    mel_pos = enc_pos[:seq]
    chd_pos = dec_pos[:seq]

    mesh = plsc.VectorSubcoreMesh(
        core_axis_name="c", subcore_axis_name="s", num_cores=2, num_subcores=16
    )
    run = pl.kernel(
        _sc_body,
        out_type=(
            jax.ShapeDtypeStruct((total, _D), jnp.float32),
            jax.ShapeDtypeStruct((total, _D), jnp.float32),
        ),
        mesh=mesh,
        scratch_types=[
            pltpu.VMEM((_CHUNK,), jnp.int32),
            pltpu.VMEM((_CHUNK,), jnp.int32),
            pltpu.VMEM((_CHUNK, _D), jnp.float32),
            pltpu.VMEM((_CHUNK, _D), jnp.float32),
            pltpu.VMEM((_SEQ, _D), jnp.float32),
            pltpu.SemaphoreType.DMA,
            pltpu.SemaphoreType.DMA,
        ],
        compiler_params=pltpu.CompilerParams(use_tc_tiling_on_sc=False),
    )
    mel_out, chd_out = run(mel_idx, chd_idx, melody_emb, chord_emb, mel_pos, chd_pos)
    return (mel_out.reshape(batch, seq, _D), chd_out.reshape(batch, seq, _D))
